# Initial kernel scaffold; baseline (speedup 1.0000x reference)
import jax
import jax.numpy as jnp
from jax.experimental import pallas as pl


def kernel(inputs, Wg, bg, We, be):
    raise NotImplementedError("write your pallas kernel here")


# speedup vs baseline: 1.7867x; 1.7867x over previous
import functools

import jax
import jax.numpy as jnp
from jax.experimental import pallas as pl
from jax.experimental.pallas import tpu as pltpu

B, S, D, E, K, OUT = 4, 4096, 1024, 8, 2, 1024
T = 16384
TT = 512


def _moe_dense_body(x_ref, wg_ref, bg_ref, we_ref, be_ref, out_ref):
    x = x_ref[...]
    logits = jnp.dot(x, wg_ref[...], preferred_element_type=jnp.float32)
    logits = logits + bg_ref[...]
    iota = jax.lax.broadcasted_iota(jnp.int32, (TT, E), 1)
    m1 = jnp.max(logits, axis=1, keepdims=True)
    i1 = jnp.min(jnp.where(logits == m1, iota, E), axis=1, keepdims=True)
    masked = jnp.where(iota == i1, jnp.finfo(jnp.float32).min, logits)
    m2 = jnp.max(masked, axis=1, keepdims=True)
    i2 = jnp.min(jnp.where(masked == m2, iota, E), axis=1, keepdims=True)
    w1 = 1.0 / (1.0 + jnp.exp(m2 - m1))
    w2 = 1.0 - w1
    wdense = jnp.where(iota == i1, w1, 0.0) + jnp.where(iota == i2, w2, 0.0)
    acc = jnp.zeros((TT, OUT), jnp.float32)
    for e in range(E):
        expert = jnp.dot(x, we_ref[e], preferred_element_type=jnp.float32)
        expert = expert + be_ref[e:e + 1, :]
        acc = acc + wdense[:, e:e + 1] * expert
    out_ref[...] = acc


@jax.jit
def _moe_dense(x2d, wg, bg2d, we, be):
    return pl.pallas_call(
        _moe_dense_body,
        grid=(T // TT,),
        in_specs=[
            pl.BlockSpec((TT, D), lambda t: (t, 0)),
            pl.BlockSpec((D, E), lambda t: (0, 0)),
            pl.BlockSpec((1, E), lambda t: (0, 0)),
            pl.BlockSpec((E, D, OUT), lambda t: (0, 0, 0)),
            pl.BlockSpec((E, OUT), lambda t: (0, 0)),
        ],
        out_specs=pl.BlockSpec((TT, OUT), lambda t: (t, 0)),
        out_shape=jax.ShapeDtypeStruct((T, OUT), jnp.float32),
    )(x2d, wg, bg2d, we, be)


def kernel(inputs, Wg, bg, We, be):
    x2d = inputs.reshape(T, D)
    out = _moe_dense(x2d, Wg, bg.reshape(1, E), We, be)
    return out.reshape(B, S, OUT)


# speedup vs baseline: 1.8204x; 1.0189x over previous
import functools

import jax
import jax.numpy as jnp
from jax.experimental import pallas as pl
from jax.experimental.pallas import tpu as pltpu

B, S, D, E, K, OUT = 4, 4096, 1024, 8, 2, 1024
T = 16384
TT = 512


def _moe_dense_body(x_ref, wg_ref, bg_ref, we_ref, be_ref, out_ref):
    x = x_ref[...]
    logits = jnp.dot(x, wg_ref[...], preferred_element_type=jnp.float32)
    logits = logits + bg_ref[...]
    iota = jax.lax.broadcasted_iota(jnp.int32, (TT, E), 1)
    m1 = jnp.max(logits, axis=1, keepdims=True)
    i1 = jnp.min(jnp.where(logits == m1, iota, E), axis=1, keepdims=True)
    masked = jnp.where(iota == i1, jnp.finfo(jnp.float32).min, logits)
    m2 = jnp.max(masked, axis=1, keepdims=True)
    i2 = jnp.min(jnp.where(masked == m2, iota, E), axis=1, keepdims=True)
    w1 = 1.0 / (1.0 + jnp.exp(m2 - m1))
    w2 = 1.0 - w1
    wdense = jnp.where(iota == i1, w1, 0.0) + jnp.where(iota == i2, w2, 0.0)
    xb = x.astype(jnp.bfloat16)
    acc = jnp.zeros((TT, OUT), jnp.float32)
    for e in range(E):
        expert = jnp.dot(xb, we_ref[e].astype(jnp.bfloat16),
                         preferred_element_type=jnp.float32)
        expert = expert + be_ref[e:e + 1, :]
        acc = acc + wdense[:, e:e + 1] * expert
    out_ref[...] = acc


@jax.jit
def _moe_dense(x2d, wg, bg2d, we, be):
    return pl.pallas_call(
        _moe_dense_body,
        grid=(T // TT,),
        in_specs=[
            pl.BlockSpec((TT, D), lambda t: (t, 0)),
            pl.BlockSpec((D, E), lambda t: (0, 0)),
            pl.BlockSpec((1, E), lambda t: (0, 0)),
            pl.BlockSpec((E, D, OUT), lambda t: (0, 0, 0)),
            pl.BlockSpec((E, OUT), lambda t: (0, 0)),
        ],
        out_specs=pl.BlockSpec((TT, OUT), lambda t: (t, 0)),
        out_shape=jax.ShapeDtypeStruct((T, OUT), jnp.float32),
    )(x2d, wg, bg2d, we, be)


def kernel(inputs, Wg, bg, We, be):
    x2d = inputs.reshape(T, D)
    out = _moe_dense(x2d, Wg, bg.reshape(1, E), We, be)
    return out.reshape(B, S, OUT)
